# baseline (speedup 1.0000x reference)
import jax
import jax.numpy as jnp
from jax.experimental import pallas as pl
from jax.experimental.pallas import tpu as pltpu


def _fuse_body(sim_ref, base_ref, p_ref, r_ref, out_ref, alpha_ref):
    x = sim_ref[:, 0, 0, :]
    G, S = x.shape
    iota = jax.lax.broadcasted_iota(jnp.int32, (G, S), 1)
    acc = jnp.zeros((G, 1), jnp.float32)
    for i in range(5):
        m = jnp.max(x, axis=-1, keepdims=True)
        acc = acc + m
        if i < 4:
            eq = x == m
            first = jnp.min(jnp.where(eq, iota, S), axis=-1, keepdims=True)
            x = jnp.where(iota == first, -jnp.inf, x)
    conf = jnp.sum(acc) / (5.0 * G)
    z = base_ref[0, 0] - conf * 10.0
    a = 1.0 / (1.0 + jnp.exp(-z))
    a = jnp.clip(a, 0.1, 0.9)
    b = pl.program_id(0)
    alpha_ref[pl.ds(b, 1)] = jnp.broadcast_to(a, (1, 1, 1))
    out_ref[...] = a * p_ref[...] + (1.0 - a) * r_ref[...]


TB = 2


def kernel(param_pred, retrieval_pred, retrieval_sim, base_alpha):
    B, T, D = param_pred.shape
    G, _, S = retrieval_sim.shape
    sim4 = retrieval_sim.reshape(G, B, 1, S)
    base = jnp.reshape(base_alpha, (1, 1)).astype(jnp.float32)

    fused, alpha = pl.pallas_call(
        _fuse_body,
        grid=(B, TB),
        in_specs=[
            pl.BlockSpec((G, 1, 1, S), lambda b, t: (0, b, 0, 0)),
            pl.BlockSpec((1, 1), lambda b, t: (0, 0)),
            pl.BlockSpec((1, T // TB, D), lambda b, t: (b, t, 0)),
            pl.BlockSpec((1, T // TB, D), lambda b, t: (b, t, 0)),
        ],
        out_specs=[
            pl.BlockSpec((1, T // TB, D), lambda b, t: (b, t, 0)),
            pl.BlockSpec((B, 1, 1), lambda b, t: (0, 0, 0)),
        ],
        out_shape=[
            jax.ShapeDtypeStruct((B, T, D), jnp.float32),
            jax.ShapeDtypeStruct((B, 1, 1), jnp.float32),
        ],
        compiler_params=pltpu.CompilerParams(
            dimension_semantics=("arbitrary", "arbitrary"),
        ),
    )(sim4, base, param_pred, retrieval_pred)
    return fused, alpha.reshape(B)


# speedup vs baseline: 2.0582x; 2.0582x over previous
import jax
import jax.numpy as jnp
from jax.experimental import pallas as pl
from jax.experimental.pallas import tpu as pltpu

BB = 2


def _fuse_body(sim_ref, base_ref, p_ref, r_ref, out_ref, alpha_ref):
    x = sim_ref[:, :, 0, :]
    G, Bb, S = x.shape
    iota = jax.lax.broadcasted_iota(jnp.int32, (G, Bb, S), 2)
    acc = jnp.zeros((G, Bb, 1), jnp.float32)
    for i in range(5):
        m = jnp.max(x, axis=-1, keepdims=True)
        acc = acc + m
        if i < 4:
            eq = x == m
            first = jnp.min(jnp.where(eq, iota, S), axis=-1, keepdims=True)
            x = jnp.where(iota == first, -jnp.inf, x)
    conf = jnp.sum(acc, axis=(0, 2)) / (5.0 * G)
    z = base_ref[0, 0] - conf * 10.0
    a = 1.0 / (1.0 + jnp.exp(-z))
    a = jnp.clip(a, 0.1, 0.9)
    b = pl.program_id(0)
    alpha_ref[pl.ds(b * Bb, Bb)] = a.reshape(Bb, 1, 1)
    a3 = a.reshape(Bb, 1, 1)
    out_ref[...] = a3 * p_ref[...] + (1.0 - a3) * r_ref[...]


def kernel(param_pred, retrieval_pred, retrieval_sim, base_alpha):
    B, T, D = param_pred.shape
    G, _, S = retrieval_sim.shape
    sim4 = retrieval_sim.reshape(G, B, 1, S)
    base = jnp.reshape(base_alpha, (1, 1)).astype(jnp.float32)

    fused, alpha = pl.pallas_call(
        _fuse_body,
        grid=(B // BB,),
        in_specs=[
            pl.BlockSpec((G, BB, 1, S), lambda b: (0, b, 0, 0)),
            pl.BlockSpec((1, 1), lambda b: (0, 0)),
            pl.BlockSpec((BB, T, D), lambda b: (b, 0, 0)),
            pl.BlockSpec((BB, T, D), lambda b: (b, 0, 0)),
        ],
        out_specs=[
            pl.BlockSpec((BB, T, D), lambda b: (b, 0, 0)),
            pl.BlockSpec((B, 1, 1), lambda b: (0, 0, 0)),
        ],
        out_shape=[
            jax.ShapeDtypeStruct((B, T, D), jnp.float32),
            jax.ShapeDtypeStruct((B, 1, 1), jnp.float32),
        ],
        compiler_params=pltpu.CompilerParams(
            dimension_semantics=("arbitrary",),
        ),
    )(sim4, base, param_pred, retrieval_pred)
    return fused, alpha.reshape(B)


# speedup vs baseline: 2.2432x; 1.0899x over previous
import jax
import jax.numpy as jnp
from jax.experimental import pallas as pl
from jax.experimental.pallas import tpu as pltpu

BB = 4


def _fuse_body(sim_ref, base_ref, p_ref, r_ref, out_ref, alpha_ref):
    x = sim_ref[:, :, 0, :]
    G, Bb, S = x.shape
    iota = jax.lax.broadcasted_iota(jnp.int32, (G, Bb, S), 2)
    acc = jnp.zeros((G, Bb, 1), jnp.float32)
    for i in range(5):
        m = jnp.max(x, axis=-1, keepdims=True)
        acc = acc + m
        if i < 4:
            eq = x == m
            first = jnp.min(jnp.where(eq, iota, S), axis=-1, keepdims=True)
            x = jnp.where(iota == first, -jnp.inf, x)
    conf = jnp.sum(acc, axis=(0, 2)) / (5.0 * G)
    z = base_ref[0, 0] - conf * 10.0
    a = 1.0 / (1.0 + jnp.exp(-z))
    a = jnp.clip(a, 0.1, 0.9)
    b = pl.program_id(0)
    alpha_ref[pl.ds(b * Bb, Bb)] = a.reshape(Bb, 1, 1)
    a3 = a.reshape(Bb, 1, 1)
    out_ref[...] = a3 * p_ref[...] + (1.0 - a3) * r_ref[...]


def kernel(param_pred, retrieval_pred, retrieval_sim, base_alpha):
    B, T, D = param_pred.shape
    G, _, S = retrieval_sim.shape
    sim4 = retrieval_sim.reshape(G, B, 1, S)
    base = jnp.reshape(base_alpha, (1, 1)).astype(jnp.float32)

    fused, alpha = pl.pallas_call(
        _fuse_body,
        grid=(B // BB,),
        in_specs=[
            pl.BlockSpec((G, BB, 1, S), lambda b: (0, b, 0, 0)),
            pl.BlockSpec((1, 1), lambda b: (0, 0)),
            pl.BlockSpec((BB, T, D), lambda b: (b, 0, 0)),
            pl.BlockSpec((BB, T, D), lambda b: (b, 0, 0)),
        ],
        out_specs=[
            pl.BlockSpec((BB, T, D), lambda b: (b, 0, 0)),
            pl.BlockSpec((B, 1, 1), lambda b: (0, 0, 0)),
        ],
        out_shape=[
            jax.ShapeDtypeStruct((B, T, D), jnp.float32),
            jax.ShapeDtypeStruct((B, 1, 1), jnp.float32),
        ],
        compiler_params=pltpu.CompilerParams(
            dimension_semantics=("arbitrary",),
        ),
    )(sim4, base, param_pred, retrieval_pred)
    return fused, alpha.reshape(B)
